# baseline (speedup 1.0000x reference)
import functools

import jax
import jax.numpy as jnp
from jax import lax
from jax.experimental import pallas as pl
from jax.experimental.pallas import tpu as pltpu
from jax.experimental.pallas import tpu_sc as plsc

NUM_ROWS = 1000000
FEAT = 64
BATCH = 4096
SEQ = 200
LANES = 16
NCHUNK = FEAT // LANES
NUM_CORES = 2
NUM_SUBCORES = 16
NUM_WORKERS = NUM_CORES * NUM_SUBCORES
SEQ_PER_W = BATCH // NUM_WORKERS
GCHUNK_A = 128
GCHUNK_B = SEQ - GCHUNK_A


def _rsqrt_newton(x):
    i = lax.bitcast_convert_type(x, jnp.int32)
    i = jnp.int32(0x5F3759DF) - lax.shift_right_logical(i, 1)
    y = lax.bitcast_convert_type(i, jnp.float32)
    for _ in range(3):
        y = y * (1.5 - (0.5 * x) * y * y)
    return y


def _make_sc_kernel():
    mesh = plsc.VectorSubcoreMesh(core_axis_name="c", subcore_axis_name="s")

    @functools.partial(
        pl.kernel,
        mesh=mesh,
        compiler_params=pltpu.CompilerParams(use_tc_tiling_on_sc=False),
        out_type=jax.ShapeDtypeStruct((BATCH * SEQ, FEAT), jnp.float32),
        scratch_types=[
            pltpu.VMEM((SEQ_PER_W * SEQ,), jnp.int32),
            pltpu.VMEM((SEQ, FEAT), jnp.float32),
            pltpu.SemaphoreType.DMA,
        ],
    )
    def sc_kernel(idx_hbm, table_hbm, out_hbm, idx_v, buf, gsem):
        wid = lax.axis_index("s") * NUM_CORES + lax.axis_index("c")
        base = wid * SEQ_PER_W

        pltpu.sync_copy(idx_hbm.at[pl.ds(base * SEQ, SEQ_PER_W * SEQ)], idx_v)

        inv_n = jnp.float32(1.0 / SEQ)
        inv_nm1 = jnp.float32(1.0 / (SEQ - 1))
        zero = jnp.zeros((LANES,), jnp.float32)

        @pl.loop(0, SEQ_PER_W)
        def _(s):
            off = s * SEQ
            pltpu.async_copy(
                table_hbm.at[idx_v.at[pl.ds(off, GCHUNK_A)]],
                buf.at[pl.ds(0, GCHUNK_A)],
                gsem,
            )
            pltpu.async_copy(
                table_hbm.at[idx_v.at[pl.ds(off + GCHUNK_A, GCHUNK_B)]],
                buf.at[pl.ds(GCHUNK_A, GCHUNK_B)],
                gsem,
            ).wait()
            pltpu.make_async_copy(
                table_hbm.at[idx_v.at[pl.ds(off, GCHUNK_A)]],
                buf.at[pl.ds(0, GCHUNK_A)],
                gsem,
            ).wait()

            def p1(r, carry):
                out = []
                for c in range(NCHUNK):
                    v = buf[r, pl.ds(LANES * c, LANES)]
                    out.append(carry[2 * c] + v)
                    out.append(carry[2 * c + 1] + v * v)
                return tuple(out)

            acc = lax.fori_loop(0, SEQ, p1, (zero,) * (2 * NCHUNK))

            scale = []
            shift = []
            for c in range(NCHUNK):
                sm = acc[2 * c]
                sq = acc[2 * c + 1]
                mean = sm * inv_n
                var = jnp.maximum((sq - sm * mean) * inv_nm1, 0.0)
                std = var * _rsqrt_newton(var)
                inv = 1.0 / (std + 1e-8)
                scale.append(inv)
                shift.append(-mean * inv)

            def p2(r, carry):
                for c in range(NCHUNK):
                    v = buf[r, pl.ds(LANES * c, LANES)]
                    buf[r, pl.ds(LANES * c, LANES)] = v * scale[c] + shift[c]
                return carry

            lax.fori_loop(0, SEQ, p2, 0)

            pltpu.sync_copy(buf, out_hbm.at[pl.ds((base + s) * SEQ, SEQ)])

    return sc_kernel


_SC_KERNEL = _make_sc_kernel()


@jax.jit
def kernel(index_tensor, features):
    idx = index_tensor.astype(jnp.int32).reshape(-1)
    out = _SC_KERNEL(idx, features)
    return out.reshape(BATCH, SEQ, FEAT)


# speedup vs baseline: 1.1990x; 1.1990x over previous
import functools

import jax
import jax.numpy as jnp
from jax import lax
from jax.experimental import pallas as pl
from jax.experimental.pallas import tpu as pltpu
from jax.experimental.pallas import tpu_sc as plsc

NUM_ROWS = 1000000
FEAT = 64
BATCH = 4096
SEQ = 200
LANES = 16
NCHUNK = FEAT // LANES
NUM_CORES = 2
NUM_SUBCORES = 16
NUM_WORKERS = NUM_CORES * NUM_SUBCORES
SEQ_PER_W = BATCH // NUM_WORKERS
GCHUNK_A = 128
GCHUNK_B = SEQ - GCHUNK_A


def _rsqrt_newton(x):
    i = lax.bitcast_convert_type(x, jnp.int32)
    i = jnp.int32(0x5F3759DF) - lax.shift_right_logical(i, 1)
    y = lax.bitcast_convert_type(i, jnp.float32)
    for _ in range(3):
        y = y * (1.5 - (0.5 * x) * y * y)
    return y


def _make_sc_kernel():
    mesh = plsc.VectorSubcoreMesh(core_axis_name="c", subcore_axis_name="s")

    @functools.partial(
        pl.kernel,
        mesh=mesh,
        compiler_params=pltpu.CompilerParams(use_tc_tiling_on_sc=False),
        out_type=jax.ShapeDtypeStruct((BATCH * SEQ, FEAT), jnp.float32),
        scratch_types=[
            pltpu.VMEM((SEQ_PER_W * SEQ,), jnp.int32),
            pltpu.VMEM((SEQ, FEAT), jnp.float32),
            pltpu.VMEM((SEQ, FEAT), jnp.float32),
            pltpu.VMEM((SEQ, FEAT), jnp.float32),
            pltpu.VMEM((SEQ, FEAT), jnp.float32),
            pltpu.SemaphoreType.DMA,
            pltpu.SemaphoreType.DMA,
            pltpu.SemaphoreType.DMA,
            pltpu.SemaphoreType.DMA,
            pltpu.SemaphoreType.DMA,
            pltpu.SemaphoreType.DMA,
            pltpu.SemaphoreType.DMA,
            pltpu.SemaphoreType.DMA,
        ],
    )
    def sc_kernel(idx_hbm, table_hbm, out_hbm, idx_v,
                  b0, b1, b2, b3, g0, g1, g2, g3, w0, w1, w2, w3):
        bufs = (b0, b1, b2, b3)
        gsems = (g0, g1, g2, g3)
        wsems = (w0, w1, w2, w3)

        wid = lax.axis_index("s") * NUM_CORES + lax.axis_index("c")
        base = wid * SEQ_PER_W

        pltpu.sync_copy(idx_hbm.at[pl.ds(base * SEQ, SEQ_PER_W * SEQ)], idx_v)

        inv_n = jnp.float32(1.0 / SEQ)
        inv_nm1 = jnp.float32(1.0 / (SEQ - 1))
        zero = jnp.zeros((LANES,), jnp.float32)

        def gstart(s, buf, sem):
            off = s * SEQ
            pltpu.async_copy(
                table_hbm.at[idx_v.at[pl.ds(off, GCHUNK_A)]],
                buf.at[pl.ds(0, GCHUNK_A)],
                sem,
            )
            pltpu.async_copy(
                table_hbm.at[idx_v.at[pl.ds(off + GCHUNK_A, GCHUNK_B)]],
                buf.at[pl.ds(GCHUNK_A, GCHUNK_B)],
                sem,
            )

        def gwait(buf, sem):
            pltpu.make_async_copy(out_hbm.at[pl.ds(0, SEQ)], buf, sem).wait()

        def wstart(s, buf, sem):
            pltpu.async_copy(buf, out_hbm.at[pl.ds((base + s) * SEQ, SEQ)], sem)

        def wwait(buf, sem):
            pltpu.make_async_copy(buf, out_hbm.at[pl.ds(0, SEQ)], sem).wait()

        def compute(buf):
            def p1(r2, carry):
                r = r2 * 2
                out = list(carry)
                for rr in (r, r + 1):
                    for c in range(NCHUNK):
                        v = buf[rr, pl.ds(LANES * c, LANES)]
                        out[2 * c] = out[2 * c] + v
                        out[2 * c + 1] = out[2 * c + 1] + v * v
                return tuple(out)

            acc = lax.fori_loop(0, SEQ // 2, p1, (zero,) * (2 * NCHUNK))

            scale = []
            shift = []
            for c in range(NCHUNK):
                sm = acc[2 * c]
                sq = acc[2 * c + 1]
                mean = sm * inv_n
                var = jnp.maximum((sq - sm * mean) * inv_nm1, 0.0)
                std = var * _rsqrt_newton(var)
                inv = 1.0 / (std + 1e-8)
                scale.append(inv)
                shift.append(-mean * inv)

            def p2(r2, carry):
                r = r2 * 2
                for rr in (r, r + 1):
                    for c in range(NCHUNK):
                        v = buf[rr, pl.ds(LANES * c, LANES)]
                        buf[rr, pl.ds(LANES * c, LANES)] = v * scale[c] + shift[c]
                return carry

            lax.fori_loop(0, SEQ // 2, p2, 0)

        gstart(0, bufs[0], gsems[0])
        gstart(1, bufs[1], gsems[1])
        gstart(2, bufs[2], gsems[2])

        @pl.loop(0, SEQ_PER_W, step=4)
        def _(s):
            for b in range(4):
                pb = (b - 1) % 4
                t = s + b + 3

                @pl.when(t < SEQ_PER_W)
                def _():
                    if b == 0:
                        @pl.when(s > 0)
                        def _():
                            wwait(bufs[pb], wsems[pb])
                    else:
                        wwait(bufs[pb], wsems[pb])
                    gstart(t, bufs[pb], gsems[pb])

                gwait(bufs[b], gsems[b])
                compute(bufs[b])
                wstart(s + b, bufs[b], wsems[b])

        for b in range(4):
            wwait(bufs[b], wsems[b])

    return sc_kernel


_SC_KERNEL = _make_sc_kernel()


@jax.jit
def kernel(index_tensor, features):
    idx = index_tensor.astype(jnp.int32).reshape(-1)
    out = _SC_KERNEL(idx, features)
    return out.reshape(BATCH, SEQ, FEAT)
